# baseline (speedup 1.0000x reference)
import functools

import jax
import jax.numpy as jnp
from jax import lax
from jax.experimental import pallas as pl
from jax.experimental.pallas import tpu as pltpu
from jax.experimental.pallas import tpu_sc as plsc

_VOCAB = 100000
_DIM = 128
_RANK = 16
_NUM_LORAS = 4
_NUM_SEG = 8
_BATCH = 4096

_NC, _NS, _L = 2, 16, 16
_NW = _NC * _NS
_TPW = _BATCH // _NW
_G = _TPW // _L
_HOT = _NUM_LORAS * _RANK

def _take16(arr, idx):
    return arr.at[idx].get(mode="promise_in_bounds")


def _splat16(arr, j):
    return _take16(arr, jnp.full((_L,), j, dtype=jnp.int32))


@functools.cache
def _build_sc_base():
  mesh = plsc.VectorSubcoreMesh(
      core_axis_name="c", subcore_axis_name="s",
      num_cores=_NC, num_subcores=_NS,
  )

  @functools.partial(
    pl.kernel,
    out_type=jax.ShapeDtypeStruct((_BATCH, _DIM), jnp.float32),
    mesh=mesh,
    scratch_types=[
        pltpu.VMEM((_TPW,), jnp.int32),
        pltpu.VMEM((_TPW, _DIM), jnp.float32),
        pltpu.SemaphoreType.DMA,
    ],
  )
  def _sc_base(ids_hbm, w_hbm, wrows_out, idx_v, w_rows, sem_w):
    wid = lax.axis_index("s") * _NC + lax.axis_index("c")
    base = wid * _TPW
    pltpu.sync_copy(ids_hbm.at[pl.ds(base, _TPW)], idx_v)
    pltpu.async_copy(w_hbm.at[idx_v], w_rows, sem_w).wait()
    pltpu.sync_copy(w_rows, wrows_out.at[pl.ds(base, _TPW)])

  return _sc_base


@functools.cache
def _build_sc_lora():
  mesh = plsc.VectorSubcoreMesh(
      core_axis_name="c", subcore_axis_name="s",
      num_cores=_NC, num_subcores=_NS,
  )

  @functools.partial(
    pl.kernel,
    out_type=jax.ShapeDtypeStruct((_BATCH, _HOT), jnp.float32),
    mesh=mesh,
    scratch_types=[
        pltpu.VMEM((_TPW,), jnp.int32),
        pltpu.VMEM((_L,), jnp.int32),
        pltpu.VMEM((_L,), jnp.int32),
        pltpu.VMEM((_L,), jnp.float32),
        pltpu.VMEM((_TPW + _L,), jnp.int32),
        pltpu.VMEM((_TPW + _L,), jnp.int32),
        pltpu.VMEM((_TPW + _L,), jnp.float32),
        pltpu.VMEM((_G * _RANK, _L), jnp.int32),
        pltpu.VMEM((_RANK * _L, _DIM), jnp.float32),
        pltpu.VMEM((_RANK * _L, _DIM), jnp.float32),
        pltpu.VMEM((_TPW, _HOT), jnp.float32),
        pltpu.SemaphoreType.DMA,
        pltpu.SemaphoreType.DMA,
    ],
  )
  def _sc_lora(ids_hbm, sp_hbm, wi_hbm, sc_hbm, a_hbm, ahot_out,
               idx_v, sp_v, wi_v, sc_v, idxp_v, lt_v, st_v, aidx_v,
               stg0, stg1, ahot_v, sem_a0, sem_a1):
    wid = lax.axis_index("s") * _NC + lax.axis_index("c")
    base = wid * _TPW
    pltpu.sync_copy(ids_hbm.at[pl.ds(base, _TPW)], idx_v)
    pltpu.sync_copy(sp_hbm, sp_v)
    pltpu.sync_copy(wi_hbm, wi_v)
    pltpu.sync_copy(sc_hbm, sc_v)

    spv = sp_v[...]
    wiv = wi_v[...]
    scv = sc_v[...]
    for g in range(_G):
        ids = idx_v[pl.ds(g * _L, _L)]
        pos = base + g * _L + lax.iota(jnp.int32, _L)
        seg = jnp.zeros((_L,), jnp.int32)
        for j in range(1, _NUM_SEG + 1):
            seg = seg + jnp.where(_splat16(spv, j) <= pos, 1, 0)
        l = _take16(wiv, seg)
        s = _take16(scv, l) * jnp.where(l != 0, 1.0, 0.0)
        idxp_v[pl.ds(g * _L, _L)] = ids
        lt_v[pl.ds(g * _L, _L)] = l
        st_v[pl.ds(g * _L, _L)] = s
        lr = l * _RANK
        for r in range(_RANK):
            aidx_v[g * _RANK + r, :] = lax.shift_right_logical(
                (lr + r) * _VOCAB + ids, 7)

    def zbody(i):
        for k in range(_HOT // _L):
            ahot_v[i, pl.ds(k * _L, _L)] = jnp.zeros((_L,), jnp.float32)
    pl.loop(0, _TPW)(zbody)

    stgs = (stg0, stg1)
    sems = (sem_a0, sem_a1)

    def fire(g):
        buf = stgs[g % 2]
        sem = sems[g % 2]
        return [
            pltpu.async_copy(a_hbm.at[aidx_v.at[g * _RANK + r]],
                             buf.at[pl.ds(r * _L, _L)], sem)
            for r in range(_RANK)
        ]

    def make_ebody(g):
        buf = stgs[g % 2]

        def ebody(t):
            tg = g * _L + t
            idv = idxp_v[pl.ds(tg, _L)]
            ltv = lt_v[pl.ds(tg, _L)]
            stv = st_v[pl.ds(tg, _L)]
            id_s = idv[0]
            l_s = ltv[0]
            s_spl = _take16(stv, jnp.zeros((_L,), jnp.int32))
            acc = jnp.zeros((_L,), jnp.float32)
            for r in range(_RANK):
                col = (id_s + 32 * r) & 127
                vv = buf[r * _L + t, pl.ds(col & 112, _L)]
                a_spl = _take16(vv, jnp.full((_L,), col & 15, jnp.int32))
                e_r = jnp.where(lax.iota(jnp.int32, _L) == r, 1.0, 0.0)
                acc = acc + a_spl * e_r
            ahot_v[tg, pl.ds(l_s * _RANK, _L)] = acc * s_spl

        return ebody

    cps = fire(0)
    for g in range(_G):
        nxt = fire(g + 1) if g + 1 < _G else None
        for c in cps:
            c.wait()
        pl.loop(0, _L)(make_ebody(g))
        cps = nxt

    pltpu.sync_copy(ahot_v, ahot_out.at[pl.ds(base, _TPW)])

  return _sc_lora


_BLK = 512


def _tc_body(b_ref, a_ref, w_ref, o_ref):
    o_ref[...] = w_ref[...] + jnp.dot(a_ref[...], b_ref[...],
                                      preferred_element_type=jnp.float32)


def kernel(input_ids, weight, A_buffer, B_buffer, scalings, seg_indptr,
           weight_indices):
    sp16 = jnp.zeros((_L,), jnp.int32).at[: _NUM_SEG + 1].set(seg_indptr)
    wi16 = jnp.zeros((_L,), jnp.int32).at[: _NUM_SEG].set(weight_indices)
    sc16 = jnp.zeros((_L,), jnp.float32).at[: _NUM_LORAS].set(scalings)
    a_lin = jnp.transpose(A_buffer, (0, 2, 1)).reshape(
        _NUM_LORAS * _RANK * _VOCAB // _DIM, _DIM)
    b_cat = jnp.transpose(B_buffer, (0, 2, 1)).reshape(_HOT, _DIM)

    w_rows = _build_sc_base()(input_ids, weight)
    a_hot = _build_sc_lora()(input_ids, sp16, wi16, sc16, a_lin)

    out = pl.pallas_call(
        _tc_body,
        grid=(_BATCH // _BLK,),
        in_specs=[
            pl.BlockSpec((_HOT, _DIM), lambda i: (0, 0)),
            pl.BlockSpec((_BLK, _HOT), lambda i: (i, 0)),
            pl.BlockSpec((_BLK, _DIM), lambda i: (i, 0)),
        ],
        out_specs=pl.BlockSpec((_BLK, _DIM), lambda i: (i, 0)),
        out_shape=jax.ShapeDtypeStruct((_BATCH, _DIM), jnp.float32),
    )(b_cat, a_hot, w_rows)
    return out


# speedup vs baseline: 1.0108x; 1.0108x over previous
import functools

import jax
import jax.numpy as jnp
from jax import lax
from jax.experimental import pallas as pl
from jax.experimental.pallas import tpu as pltpu
from jax.experimental.pallas import tpu_sc as plsc

_VOCAB = 100000
_DIM = 128
_RANK = 16
_NUM_LORAS = 4
_NUM_SEG = 8
_BATCH = 4096

_NC, _NS, _L = 2, 16, 16
_NW = _NC * _NS
_TPW = _BATCH // _NW
_G = _TPW // _L
_HOT = _NUM_LORAS * _RANK

def _take16(arr, idx):
    return arr.at[idx].get(mode="promise_in_bounds")


def _splat16(arr, j):
    return _take16(arr, jnp.full((_L,), j, dtype=jnp.int32))


@functools.cache
def _build_sc_base():
  mesh = plsc.VectorSubcoreMesh(
      core_axis_name="c", subcore_axis_name="s",
      num_cores=_NC, num_subcores=_NS,
  )

  @functools.partial(
    pl.kernel,
    out_type=jax.ShapeDtypeStruct((_BATCH, _DIM), jnp.float32),
    mesh=mesh,
    scratch_types=[
        pltpu.VMEM((_TPW,), jnp.int32),
        pltpu.VMEM((_TPW, _DIM), jnp.float32),
        pltpu.SemaphoreType.DMA,
    ],
  )
  def _sc_base(ids_hbm, w_hbm, wrows_out, idx_v, w_rows, sem_w):
    wid = lax.axis_index("s") * _NC + lax.axis_index("c")
    base = wid * _TPW
    pltpu.sync_copy(ids_hbm.at[pl.ds(base, _TPW)], idx_v)
    pltpu.async_copy(w_hbm.at[idx_v], w_rows, sem_w).wait()
    pltpu.sync_copy(w_rows, wrows_out.at[pl.ds(base, _TPW)])

  return _sc_base


@functools.cache
def _build_sc_lora():
  mesh = plsc.VectorSubcoreMesh(
      core_axis_name="c", subcore_axis_name="s",
      num_cores=_NC, num_subcores=_NS,
  )

  @functools.partial(
    pl.kernel,
    out_type=jax.ShapeDtypeStruct((_BATCH, _HOT), jnp.float32),
    mesh=mesh,
    scratch_types=[
        pltpu.VMEM((_TPW,), jnp.int32),
        pltpu.VMEM((_L,), jnp.int32),
        pltpu.VMEM((_L,), jnp.int32),
        pltpu.VMEM((_L,), jnp.float32),
        pltpu.VMEM((_TPW + _L,), jnp.int32),
        pltpu.VMEM((_TPW + _L,), jnp.int32),
        pltpu.VMEM((_TPW + _L,), jnp.float32),
        pltpu.VMEM((_G * _RANK, _L), jnp.int32),
        pltpu.VMEM((_RANK * _L, _DIM), jnp.float32),
        pltpu.VMEM((_RANK * _L, _DIM), jnp.float32),
        pltpu.VMEM((_TPW, _HOT), jnp.float32),
        pltpu.SemaphoreType.DMA,
        pltpu.SemaphoreType.DMA,
    ],
  )
  def _sc_lora(ids_hbm, sp_hbm, wi_hbm, sc_hbm, a_hbm, ahot_out,
               idx_v, sp_v, wi_v, sc_v, idxp_v, lt_v, st_v, aidx_v,
               stg0, stg1, ahot_v, sem_a0, sem_a1):
    wid = lax.axis_index("s") * _NC + lax.axis_index("c")
    base = wid * _TPW
    pltpu.sync_copy(ids_hbm.at[pl.ds(base, _TPW)], idx_v)
    pltpu.sync_copy(sp_hbm, sp_v)
    pltpu.sync_copy(wi_hbm, wi_v)
    pltpu.sync_copy(sc_hbm, sc_v)

    spv = sp_v[...]
    wiv = wi_v[...]
    scv = sc_v[...]
    for g in range(_G):
        ids = idx_v[pl.ds(g * _L, _L)]
        pos = base + g * _L + lax.iota(jnp.int32, _L)
        seg = jnp.zeros((_L,), jnp.int32)
        for j in range(1, _NUM_SEG + 1):
            seg = seg + jnp.where(_splat16(spv, j) <= pos, 1, 0)
        l = _take16(wiv, seg)
        s = _take16(scv, l) * jnp.where(l != 0, 1.0, 0.0)
        idxp_v[pl.ds(g * _L, _L)] = ids
        lt_v[pl.ds(g * _L, _L)] = l
        st_v[pl.ds(g * _L, _L)] = s
        lr = l * _RANK
        for r in range(_RANK):
            aidx_v[g * _RANK + r, :] = lax.shift_right_logical(
                (lr + r) * _VOCAB + ids, 7)

    def zbody(i):
        for k in range(_HOT // _L):
            ahot_v[i, pl.ds(k * _L, _L)] = jnp.zeros((_L,), jnp.float32)
    pl.loop(0, _TPW)(zbody)

    stgs = (stg0, stg1)
    sems = (sem_a0, sem_a1)

    def fire(g):
        buf = stgs[g % 2]
        sem = sems[g % 2]
        return [
            pltpu.async_copy(a_hbm.at[aidx_v.at[g * _RANK + r]],
                             buf.at[pl.ds(r * _L, _L)], sem)
            for r in range(_RANK)
        ]

    def make_ebody(g):
        buf = stgs[g % 2]

        def ebody(t):
            tg = g * _L + t
            idv = idxp_v[pl.ds(tg, _L)]
            ltv = lt_v[pl.ds(tg, _L)]
            stv = st_v[pl.ds(tg, _L)]
            id_s = idv[0]
            l_s = ltv[0]
            s_spl = _take16(stv, jnp.zeros((_L,), jnp.int32))
            acc = jnp.zeros((_L,), jnp.float32)
            for r in range(_RANK):
                col = (id_s + 32 * r) & 127
                vv = buf[r * _L + t, pl.ds(col & 112, _L)]
                a_spl = _take16(vv, jnp.full((_L,), col & 15, jnp.int32))
                e_r = jnp.where(lax.iota(jnp.int32, _L) == r, 1.0, 0.0)
                acc = acc + a_spl * e_r
            ahot_v[tg, pl.ds(l_s * _RANK, _L)] = acc * s_spl

        return ebody

    cps = fire(0)
    for g in range(_G):
        nxt = fire(g + 1) if g + 1 < _G else None
        for c in cps:
            c.wait()
        pl.loop(0, _L)(make_ebody(g))
        cps = nxt

    pltpu.sync_copy(ahot_v, ahot_out.at[pl.ds(base, _TPW)])

  return _sc_lora


_BLK = 512


def _tc_body(b_ref, a_ref, w_ref, o_ref):
    o_ref[...] = w_ref[...] + jnp.dot(a_ref[...], b_ref[...],
                                      preferred_element_type=jnp.float32)


def kernel(input_ids, weight, A_buffer, B_buffer, scalings, seg_indptr,
           weight_indices):
    sp16 = jnp.zeros((_L,), jnp.int32).at[: _NUM_SEG + 1].set(seg_indptr)
    wi16 = jnp.zeros((_L,), jnp.int32).at[: _NUM_SEG].set(weight_indices)
    sc16 = jnp.zeros((_L,), jnp.float32).at[: _NUM_LORAS].set(scalings)
    a_lin = jnp.transpose(A_buffer, (0, 2, 1)).reshape(
        _NUM_LORAS * _RANK * _VOCAB // _DIM, _DIM)
    b_cat = jnp.transpose(B_buffer, (0, 2, 1)).reshape(_HOT, _DIM)

    w_rows = _build_sc_base()(input_ids, weight)
    ids2 = input_ids + (w_rows[0, 0] * 0.0).astype(jnp.int32)
    a_hot = _build_sc_lora()(ids2, sp16, wi16, sc16, a_lin)

    out = pl.pallas_call(
        _tc_body,
        grid=(_BATCH // _BLK,),
        in_specs=[
            pl.BlockSpec((_HOT, _DIM), lambda i: (0, 0)),
            pl.BlockSpec((_BLK, _HOT), lambda i: (i, 0)),
            pl.BlockSpec((_BLK, _DIM), lambda i: (i, 0)),
        ],
        out_specs=pl.BlockSpec((_BLK, _DIM), lambda i: (i, 0)),
        out_shape=jax.ShapeDtypeStruct((_BATCH, _DIM), jnp.float32),
    )(b_cat, a_hot, w_rows)
    return out


# speedup vs baseline: 1.0992x; 1.0875x over previous
import functools

import jax
import jax.numpy as jnp
from jax import lax
from jax.experimental import pallas as pl
from jax.experimental.pallas import tpu as pltpu
from jax.experimental.pallas import tpu_sc as plsc

_VOCAB = 100000
_DIM = 128
_RANK = 16
_NUM_LORAS = 4
_NUM_SEG = 8
_BATCH = 4096

_NC, _NS, _L = 2, 16, 16
_NW = _NC * _NS
_TPW = _BATCH // _NW
_G = _TPW // _L
_HOT = _NUM_LORAS * _RANK

def _take16(arr, idx):
    return arr.at[idx].get(mode="promise_in_bounds")


def _splat16(arr, j):
    return _take16(arr, jnp.full((_L,), j, dtype=jnp.int32))


@functools.cache
def _build_sc_gather():
  mesh = plsc.VectorSubcoreMesh(
      core_axis_name="c", subcore_axis_name="s",
      num_cores=_NC, num_subcores=_NS,
  )

  @functools.partial(
    pl.kernel,
    out_type=(
        jax.ShapeDtypeStruct((_BATCH, _DIM), jnp.float32),
        jax.ShapeDtypeStruct((_BATCH, _HOT), jnp.float32),
    ),
    mesh=mesh,
    scratch_types=[
        pltpu.VMEM((_TPW,), jnp.int32),
        pltpu.VMEM((_L,), jnp.int32),
        pltpu.VMEM((_L,), jnp.int32),
        pltpu.VMEM((_L,), jnp.float32),
        pltpu.VMEM((_TPW + _L,), jnp.int32),
        pltpu.VMEM((_TPW + _L,), jnp.int32),
        pltpu.VMEM((_TPW + _L,), jnp.float32),
        pltpu.VMEM((_G * _RANK, _L), jnp.int32),
        pltpu.VMEM((_RANK * _L, _DIM), jnp.float32),
        pltpu.VMEM((_RANK * _L, _DIM), jnp.float32),
        pltpu.VMEM((_TPW, _HOT), jnp.float32),
        pltpu.VMEM((_TPW, _DIM), jnp.float32),
        pltpu.SemaphoreType.DMA,
        pltpu.SemaphoreType.DMA,
        pltpu.SemaphoreType.DMA,
    ],
  )
  def _sc_gather(ids_hbm, sp_hbm, wi_hbm, sc_hbm, w_hbm, a_hbm,
                 wrows_out, ahot_out,
                 idx_v, sp_v, wi_v, sc_v, idxp_v, lt_v, st_v, aidx_v,
                 stg0, stg1, ahot_v, w_rows, sem_w, sem_a0, sem_a1):
    wid = lax.axis_index("s") * _NC + lax.axis_index("c")
    base = wid * _TPW
    pltpu.sync_copy(ids_hbm.at[pl.ds(base, _TPW)], idx_v)
    pltpu.sync_copy(sp_hbm, sp_v.at[pl.ds(0, _NUM_SEG + 1)])
    pltpu.sync_copy(wi_hbm, wi_v.at[pl.ds(0, _NUM_SEG)])
    pltpu.sync_copy(sc_hbm, sc_v.at[pl.ds(0, _NUM_LORAS)])
    wcopy = pltpu.async_copy(w_hbm.at[idx_v], w_rows, sem_w)

    spv = sp_v[...]
    wiv = wi_v[...]
    scv = sc_v[...]
    for g in range(_G):
        ids = idx_v[pl.ds(g * _L, _L)]
        pos = base + g * _L + lax.iota(jnp.int32, _L)
        seg = jnp.zeros((_L,), jnp.int32)
        for j in range(1, _NUM_SEG + 1):
            seg = seg + jnp.where(_splat16(spv, j) <= pos, 1, 0)
        l = _take16(wiv, seg)
        s = _take16(scv, l) * jnp.where(l != 0, 1.0, 0.0)
        idxp_v[pl.ds(g * _L, _L)] = ids
        lt_v[pl.ds(g * _L, _L)] = l
        st_v[pl.ds(g * _L, _L)] = s
        lr = l * _RANK
        for r in range(_RANK):
            aidx_v[g * _RANK + r, :] = lax.shift_right_logical(
                (lr + r) * _VOCAB + ids, 7)

    def zbody(i):
        for k in range(_HOT // _L):
            ahot_v[i, pl.ds(k * _L, _L)] = jnp.zeros((_L,), jnp.float32)
    pl.loop(0, _TPW)(zbody)

    stgs = (stg0, stg1)
    sems = (sem_a0, sem_a1)

    def fire(g):
        buf = stgs[g % 2]
        sem = sems[g % 2]
        return [
            pltpu.async_copy(a_hbm.at[aidx_v.at[g * _RANK + r]],
                             buf.at[pl.ds(r * _L, _L)], sem)
            for r in range(_RANK)
        ]

    def make_ebody(g):
        buf = stgs[g % 2]

        def ebody(t):
            tg = g * _L + t
            idv = idxp_v[pl.ds(tg, _L)]
            ltv = lt_v[pl.ds(tg, _L)]
            stv = st_v[pl.ds(tg, _L)]
            id_s = idv[0]
            l_s = ltv[0]
            s_spl = _take16(stv, jnp.zeros((_L,), jnp.int32))
            acc = jnp.zeros((_L,), jnp.float32)
            for r in range(_RANK):
                col = (id_s + 32 * r) & 127
                vv = buf[r * _L + t, pl.ds(col & 112, _L)]
                a_spl = _take16(vv, jnp.full((_L,), col & 15, jnp.int32))
                e_r = jnp.where(lax.iota(jnp.int32, _L) == r, 1.0, 0.0)
                acc = acc + a_spl * e_r
            ahot_v[tg, pl.ds(l_s * _RANK, _L)] = acc * s_spl

        return ebody

    cps = fire(0)
    for g in range(_G):
        nxt = fire(g + 1) if g + 1 < _G else None
        for c in cps:
            c.wait()
        pl.loop(0, _L)(make_ebody(g))
        cps = nxt

    pltpu.sync_copy(ahot_v, ahot_out.at[pl.ds(base, _TPW)])
    wcopy.wait()
    pltpu.sync_copy(w_rows, wrows_out.at[pl.ds(base, _TPW)])

  return _sc_gather


_BLK = 512


def _tc_body(b_ref, a_ref, w_ref, o_ref):
    o_ref[...] = w_ref[...] + jnp.dot(a_ref[...], b_ref[...],
                                      preferred_element_type=jnp.float32)


def kernel(input_ids, weight, A_buffer, B_buffer, scalings, seg_indptr,
           weight_indices):
    a_lin = jnp.transpose(A_buffer, (0, 2, 1)).reshape(
        _NUM_LORAS * _RANK * _VOCAB // _DIM, _DIM)
    b_cat = jnp.transpose(B_buffer, (0, 2, 1)).reshape(_HOT, _DIM)

    w_rows, a_hot = _build_sc_gather()(
        input_ids, seg_indptr, weight_indices, scalings, weight, a_lin
    )

    out = pl.pallas_call(
        _tc_body,
        grid=(_BATCH // _BLK,),
        in_specs=[
            pl.BlockSpec((_HOT, _DIM), lambda i: (0, 0)),
            pl.BlockSpec((_BLK, _HOT), lambda i: (i, 0)),
            pl.BlockSpec((_BLK, _DIM), lambda i: (i, 0)),
        ],
        out_specs=pl.BlockSpec((_BLK, _DIM), lambda i: (i, 0)),
        out_shape=jax.ShapeDtypeStruct((_BATCH, _DIM), jnp.float32),
    )(b_cat, a_hot, w_rows)
    return out


# speedup vs baseline: 1.1184x; 1.0175x over previous
import functools

import jax
import jax.numpy as jnp
from jax import lax
from jax.experimental import pallas as pl
from jax.experimental.pallas import tpu as pltpu
from jax.experimental.pallas import tpu_sc as plsc

_VOCAB = 100000
_DIM = 128
_RANK = 16
_NUM_LORAS = 4
_NUM_SEG = 8
_BATCH = 4096

_NC, _NS, _L = 2, 16, 16
_NW = _NC * _NS
_TPW = _BATCH // _NW
_G = _TPW // _L
_HOT = _NUM_LORAS * _RANK

def _take16(arr, idx):
    return arr.at[idx].get(mode="promise_in_bounds")


def _splat16(arr, j):
    return _take16(arr, jnp.full((_L,), j, dtype=jnp.int32))


@functools.cache
def _build_sc_gather():
  mesh = plsc.VectorSubcoreMesh(
      core_axis_name="c", subcore_axis_name="s",
      num_cores=_NC, num_subcores=_NS,
  )

  @functools.partial(
    pl.kernel,
    out_type=(
        jax.ShapeDtypeStruct((_BATCH, _DIM), jnp.float32),
        jax.ShapeDtypeStruct((_BATCH, _HOT), jnp.float32),
    ),
    mesh=mesh,
    scratch_types=[
        pltpu.VMEM((_TPW,), jnp.int32),
        pltpu.VMEM((_L,), jnp.int32),
        pltpu.VMEM((_L,), jnp.int32),
        pltpu.VMEM((_L,), jnp.float32),
        pltpu.VMEM((_TPW + _L,), jnp.int32),
        pltpu.VMEM((_TPW + _L,), jnp.int32),
        pltpu.VMEM((_TPW + _L,), jnp.float32),
        pltpu.VMEM((_G * _RANK, _L), jnp.int32),
        pltpu.VMEM((_RANK * _L, _DIM), jnp.float32),
        pltpu.VMEM((_RANK * _L, _DIM), jnp.float32),
        pltpu.VMEM((_TPW, _HOT), jnp.float32),
        pltpu.VMEM((_TPW, _DIM), jnp.float32),
        pltpu.SemaphoreType.DMA,
        pltpu.SemaphoreType.DMA,
        pltpu.SemaphoreType.DMA,
        pltpu.SemaphoreType.DMA,
        pltpu.SemaphoreType.DMA,
    ],
  )
  def _sc_gather(ids_hbm, sp_hbm, wi_hbm, sc_hbm, w_hbm, a_hbm,
                 wrows_out, ahot_out,
                 idx_v, sp_v, wi_v, sc_v, idxp_v, lt_v, st_v, aidx_v,
                 stg0, stg1, ahot_v, w_rows, sem_w, sem_a0, sem_a1,
                 sem_in, sem_out):
    wid = lax.axis_index("s") * _NC + lax.axis_index("c")
    base = wid * _TPW
    incs = [
        pltpu.async_copy(ids_hbm.at[pl.ds(base, _TPW)], idx_v, sem_in),
        pltpu.async_copy(sp_hbm, sp_v.at[pl.ds(0, _NUM_SEG + 1)], sem_in),
        pltpu.async_copy(wi_hbm, wi_v.at[pl.ds(0, _NUM_SEG)], sem_in),
        pltpu.async_copy(sc_hbm, sc_v.at[pl.ds(0, _NUM_LORAS)], sem_in),
    ]
    for c in incs:
        c.wait()
    wcopy = pltpu.async_copy(w_hbm.at[idx_v], w_rows, sem_w)

    spv = sp_v[...]
    wiv = wi_v[...]
    scv = sc_v[...]
    for g in range(_G):
        ids = idx_v[pl.ds(g * _L, _L)]
        pos = base + g * _L + lax.iota(jnp.int32, _L)
        seg = jnp.zeros((_L,), jnp.int32)
        for j in range(1, _NUM_SEG + 1):
            seg = seg + jnp.where(_splat16(spv, j) <= pos, 1, 0)
        l = _take16(wiv, seg)
        s = _take16(scv, l) * jnp.where(l != 0, 1.0, 0.0)
        idxp_v[pl.ds(g * _L, _L)] = ids
        lt_v[pl.ds(g * _L, _L)] = l
        st_v[pl.ds(g * _L, _L)] = s
        lr = l * _RANK
        for r in range(_RANK):
            aidx_v[g * _RANK + r, :] = lax.shift_right_logical(
                (lr + r) * _VOCAB + ids, 7)

    def zbody(i):
        for k in range(_HOT // _L):
            ahot_v[i, pl.ds(k * _L, _L)] = jnp.zeros((_L,), jnp.float32)
    pl.loop(0, _TPW)(zbody)

    stgs = (stg0, stg1)
    sems = (sem_a0, sem_a1)

    def fire(g):
        buf = stgs[g % 2]
        sem = sems[g % 2]
        return [
            pltpu.async_copy(a_hbm.at[aidx_v.at[g * _RANK + r]],
                             buf.at[pl.ds(r * _L, _L)], sem)
            for r in range(_RANK)
        ]

    def make_ebody(g):
        buf = stgs[g % 2]

        def ebody(t):
            tg = g * _L + t
            idv = idxp_v[pl.ds(tg, _L)]
            ltv = lt_v[pl.ds(tg, _L)]
            stv = st_v[pl.ds(tg, _L)]
            id_s = idv[0]
            l_s = ltv[0]
            s_spl = _take16(stv, jnp.zeros((_L,), jnp.int32))
            acc = jnp.zeros((_L,), jnp.float32)
            for r in range(_RANK):
                col = (id_s + 32 * r) & 127
                vv = buf[r * _L + t, pl.ds(col & 112, _L)]
                a_spl = _take16(vv, jnp.full((_L,), col & 15, jnp.int32))
                e_r = jnp.where(lax.iota(jnp.int32, _L) == r, 1.0, 0.0)
                acc = acc + a_spl * e_r
            ahot_v[tg, pl.ds(l_s * _RANK, _L)] = acc * s_spl

        return ebody

    cps = fire(0)
    outs = []
    for g in range(_G):
        nxt = fire(g + 1) if g + 1 < _G else None
        for c in cps:
            c.wait()
        pl.loop(0, _L)(make_ebody(g))
        outs.append(pltpu.async_copy(
            ahot_v.at[pl.ds(g * _L, _L)],
            ahot_out.at[pl.ds(base + g * _L, _L)], sem_out))
        cps = nxt

    wcopy.wait()
    pltpu.sync_copy(w_rows, wrows_out.at[pl.ds(base, _TPW)])
    for c in outs:
        c.wait()

  return _sc_gather


_BLK = 512


def _tc_body(b_ref, a_ref, w_ref, o_ref):
    o_ref[...] = w_ref[...] + jnp.dot(a_ref[...], b_ref[...],
                                      preferred_element_type=jnp.float32)


def kernel(input_ids, weight, A_buffer, B_buffer, scalings, seg_indptr,
           weight_indices):
    a_lin = jnp.transpose(A_buffer, (0, 2, 1)).reshape(
        _NUM_LORAS * _RANK * _VOCAB // _DIM, _DIM)
    b_cat = jnp.transpose(B_buffer, (0, 2, 1)).reshape(_HOT, _DIM)

    w_rows, a_hot = _build_sc_gather()(
        input_ids, seg_indptr, weight_indices, scalings, weight, a_lin
    )

    out = pl.pallas_call(
        _tc_body,
        grid=(_BATCH // _BLK,),
        in_specs=[
            pl.BlockSpec((_HOT, _DIM), lambda i: (0, 0)),
            pl.BlockSpec((_BLK, _HOT), lambda i: (i, 0)),
            pl.BlockSpec((_BLK, _DIM), lambda i: (i, 0)),
        ],
        out_specs=pl.BlockSpec((_BLK, _DIM), lambda i: (i, 0)),
        out_shape=jax.ShapeDtypeStruct((_BATCH, _DIM), jnp.float32),
    )(b_cat, a_hot, w_rows)
    return out
